# baseline (speedup 1.0000x reference)
import functools

import jax
import jax.numpy as jnp
from jax import lax
from jax.experimental import pallas as pl
from jax.experimental.pallas import tpu as pltpu
from jax.experimental.pallas import tpu_sc as plsc

_NUM_CLASSES = 56
_N_CTX = 4
_N_META = 16
_SUFFIX_LEN = 60
_DIM = 512
_BATCH = 4096
_ROWS = 1 + _N_CTX + _N_META + _SUFFIX_LEN
_NBC = _ROWS - _N_CTX

_NC = 2
_NS = 16
_NW = _NC * _NS
_BPW = _BATCH // _NW
_LANES = 16

_CCH = 32
_CUN = _N_CTX * (_BPW // _CCH)
_REP = 64
_QTR = 1024
_BUN = _NBC * (_BATCH // _QTR)
_BPWK = 9


def _body(lbl_hbm, cls_hbm, rep_hbm, out_hbm,
          lbl_v, idx_v, cbuf0, cbuf1, bbuf0, bbuf1,
          gs0, gs1, cs0, cs1, ws0, ws1):
    wid = lax.axis_index("s") * _NC + lax.axis_index("c")
    base = wid * _BPW

    pltpu.sync_copy(lbl_hbm.at[pl.ds(base, _BPW)], lbl_v)

    lanes = jax.lax.iota(jnp.int32, _LANES)
    for g in range(_BPW // _LANES):
        lv = lbl_v[pl.ds(g * _LANES, _LANES)]
        for rr in range(_N_CTX):
            plsc.store_scatter(idx_v, [lanes + (rr * _BPW + g * _LANES)],
                               lv * _N_CTX + rr)

    cbufs = (cbuf0, cbuf1)
    gsems = (gs0, gs1)
    csems = (cs0, cs1)

    def cls_gather(i, buf, sem):
        off = pl.multiple_of(i * _CCH, _CCH)
        return pltpu.async_copy(cls_hbm.at[idx_v.at[pl.ds(off, _CCH)]], buf,
                                sem)

    def cls_dst(i):
        row = 1 + i // (_BPW // _CCH)
        boff = base + _CCH * (i % (_BPW // _CCH))
        return out_hbm.at[row, pl.ds(boff, _CCH)]

    for s in range(2):
        cls_gather(s, cbufs[s], gsems[s])
    for i in range(_CUN):
        s = i % 2
        pltpu.make_async_copy(cls_hbm.at[pl.ds(0, _CCH)], cbufs[s],
                              gsems[s]).wait()
        pltpu.async_copy(cbufs[s], cls_dst(i), csems[s])
        if i + 2 < _CUN:
            pltpu.make_async_copy(cbufs[s], cls_dst(i), csems[s]).wait()
            cls_gather(i + 2, cbufs[s], gsems[s])

    bbufs = (bbuf0, bbuf1)
    wsems = (ws0, ws1)
    for t in range(_BPWK):
        s = t % 2
        u = wid + _NW * t
        j = u // (_BATCH // _QTR)
        qtr = u % (_BATCH // _QTR)
        row = jnp.where(j == 0, 0, j + _N_CTX)
        if t >= 2:
            for _ in range(_QTR // _REP):
                pltpu.make_async_copy(
                    bbufs[s], out_hbm.at[0, pl.ds(0, _REP)], wsems[s]).wait()
        pltpu.sync_copy(rep_hbm.at[j], bbufs[s])
        for k in range(_QTR // _REP):
            pltpu.async_copy(bbufs[s],
                             out_hbm.at[row, pl.ds(qtr * _QTR + k * _REP,
                                                   _REP)],
                             wsems[s])

    for s in range(2):
        pltpu.make_async_copy(cbufs[s], cls_dst(0), csems[s]).wait()
        for _ in range(_QTR // _REP):
            pltpu.make_async_copy(
                bbufs[s], out_hbm.at[0, pl.ds(0, _REP)], wsems[s]).wait()

    def half_unit(uh, buf, wsem):
        q = _BPWK * _NW + uh // 2
        j = q // (_BATCH // _QTR)
        row = j + _N_CTX
        boff = (q % (_BATCH // _QTR)) * _QTR + (uh % 2) * (_QTR // 2)
        pltpu.sync_copy(rep_hbm.at[j], buf)
        for k in range(_QTR // (2 * _REP)):
            pltpu.async_copy(buf, out_hbm.at[row, pl.ds(boff + k * _REP,
                                                        _REP)], wsem)
        for _ in range(_QTR // (2 * _REP)):
            pltpu.make_async_copy(
                buf, out_hbm.at[0, pl.ds(0, _REP)], wsem).wait()

    half_unit(wid, bbuf0, ws0)

    @pl.when(wid < 2 * (_BUN - _BPWK * _NW) - _NW)
    def _second_half_unit():
        half_unit(wid + _NW, bbuf1, ws1)


_sc_call = functools.partial(
    pl.kernel,
    out_type=jax.ShapeDtypeStruct((_ROWS, _BATCH, _DIM), jnp.float32),
    mesh=plsc.VectorSubcoreMesh(core_axis_name="c", subcore_axis_name="s"),
    compiler_params=pltpu.CompilerParams(needs_layout_passes=False),
    scratch_types=[
        pltpu.VMEM((_BPW,), jnp.int32),
        pltpu.VMEM((_N_CTX * _BPW,), jnp.int32),
        pltpu.VMEM((_CCH, _DIM), jnp.float32),
        pltpu.VMEM((_CCH, _DIM), jnp.float32),
        pltpu.VMEM((_REP, _DIM), jnp.float32),
        pltpu.VMEM((_REP, _DIM), jnp.float32),
        pltpu.SemaphoreType.DMA,
        pltpu.SemaphoreType.DMA,
        pltpu.SemaphoreType.DMA,
        pltpu.SemaphoreType.DMA,
        pltpu.SemaphoreType.DMA,
        pltpu.SemaphoreType.DMA,
    ],
)(_body)


def kernel(label, cls_ctx, meta_ctx, token_prefix, token_suffix):
    lbl = label.astype(jnp.int32)
    cls2 = cls_ctx.reshape(_NUM_CLASSES * _N_CTX, _DIM)
    pre2 = token_prefix.reshape(1, _DIM)
    meta2 = meta_ctx.reshape(_N_META, _DIM)
    suf2 = token_suffix.reshape(_SUFFIX_LEN, _DIM)
    brows = jnp.concatenate([pre2, meta2, suf2], axis=0)
    rep = jnp.broadcast_to(brows[:, None, :], (_NBC, _REP, _DIM))
    rep = jnp.reshape(rep, (_NBC, _REP, _DIM))
    out = _sc_call(lbl, cls2, rep)
    return jnp.transpose(out, (1, 0, 2))


# speedup vs baseline: 1.0030x; 1.0030x over previous
import functools

import jax
import jax.numpy as jnp
from jax import lax
from jax.experimental import pallas as pl
from jax.experimental.pallas import tpu as pltpu
from jax.experimental.pallas import tpu_sc as plsc

_NUM_CLASSES = 56
_N_CTX = 4
_N_META = 16
_SUFFIX_LEN = 60
_DIM = 512
_BATCH = 4096
_ROWS = 1 + _N_CTX + _N_META + _SUFFIX_LEN
_NBC = _ROWS - _N_CTX

_NC = 2
_NS = 16
_NW = _NC * _NS
_BPW = _BATCH // _NW
_LANES = 16

_CCH = 32
_CUN = _N_CTX * (_BPW // _CCH)
_REP = 64
_QTR = 1024
_BUN = _NBC * (_BATCH // _QTR)
_BPWK = 10


def _body(lbl_hbm, cls_hbm, rep_hbm, out_hbm,
          lbl_v, idx_v, cbuf0, cbuf1, bbuf0, bbuf1,
          gs0, gs1, cs0, cs1, ws0, ws1):
    wid = lax.axis_index("s") * _NC + lax.axis_index("c")
    base = wid * _BPW

    pltpu.sync_copy(lbl_hbm.at[pl.ds(base, _BPW)], lbl_v)

    lanes = jax.lax.iota(jnp.int32, _LANES)
    for g in range(_BPW // _LANES):
        lv = lbl_v[pl.ds(g * _LANES, _LANES)]
        for rr in range(_N_CTX):
            plsc.store_scatter(idx_v, [lanes + (rr * _BPW + g * _LANES)],
                               lv * _N_CTX + rr)

    cbufs = (cbuf0, cbuf1)
    gsems = (gs0, gs1)
    csems = (cs0, cs1)

    def cls_gather(i, buf, sem):
        off = pl.multiple_of(i * _CCH, _CCH)
        return pltpu.async_copy(cls_hbm.at[idx_v.at[pl.ds(off, _CCH)]], buf,
                                sem)

    def cls_dst(i):
        row = 1 + i // (_BPW // _CCH)
        boff = base + _CCH * (i % (_BPW // _CCH))
        return out_hbm.at[row, pl.ds(boff, _CCH)]

    bbufs = (bbuf0, bbuf1)
    wsems = (ws0, ws1)

    def bcast_unit(t, drain):
        s = t % 2
        u = wid + _NW * t
        u = jnp.where(u < _BUN, u, u - _BUN)
        j = u // (_BATCH // _QTR)
        qtr = u % (_BATCH // _QTR)
        row = jnp.where(j == 0, 0, j + _N_CTX)
        if drain:
            for _ in range(_QTR // _REP):
                pltpu.make_async_copy(
                    bbufs[s], out_hbm.at[0, pl.ds(0, _REP)], wsems[s]).wait()
        pltpu.sync_copy(rep_hbm.at[j], bbufs[s])
        for k in range(_QTR // _REP):
            pltpu.async_copy(bbufs[s],
                             out_hbm.at[row, pl.ds(qtr * _QTR + k * _REP,
                                                   _REP)],
                             wsems[s])

    for t in range(2):
        bcast_unit(t, drain=False)

    for s in range(2):
        cls_gather(s, cbufs[s], gsems[s])
    for i in range(_CUN):
        s = i % 2
        pltpu.make_async_copy(cls_hbm.at[pl.ds(0, _CCH)], cbufs[s],
                              gsems[s]).wait()
        pltpu.async_copy(cbufs[s], cls_dst(i), csems[s])
        if i + 2 < _CUN:
            pltpu.make_async_copy(cbufs[s], cls_dst(i), csems[s]).wait()
            cls_gather(i + 2, cbufs[s], gsems[s])

    for t in range(2, _BPWK):
        bcast_unit(t, drain=True)

    for s in range(2):
        pltpu.make_async_copy(cbufs[s], cls_dst(0), csems[s]).wait()
        for _ in range(_QTR // _REP):
            pltpu.make_async_copy(
                bbufs[s], out_hbm.at[0, pl.ds(0, _REP)], wsems[s]).wait()


_sc_call = functools.partial(
    pl.kernel,
    out_type=jax.ShapeDtypeStruct((_ROWS, _BATCH, _DIM), jnp.float32),
    mesh=plsc.VectorSubcoreMesh(core_axis_name="c", subcore_axis_name="s"),
    compiler_params=pltpu.CompilerParams(needs_layout_passes=False),
    scratch_types=[
        pltpu.VMEM((_BPW,), jnp.int32),
        pltpu.VMEM((_N_CTX * _BPW,), jnp.int32),
        pltpu.VMEM((_CCH, _DIM), jnp.float32),
        pltpu.VMEM((_CCH, _DIM), jnp.float32),
        pltpu.VMEM((_REP, _DIM), jnp.float32),
        pltpu.VMEM((_REP, _DIM), jnp.float32),
        pltpu.SemaphoreType.DMA,
        pltpu.SemaphoreType.DMA,
        pltpu.SemaphoreType.DMA,
        pltpu.SemaphoreType.DMA,
        pltpu.SemaphoreType.DMA,
        pltpu.SemaphoreType.DMA,
    ],
)(_body)


def kernel(label, cls_ctx, meta_ctx, token_prefix, token_suffix):
    lbl = label.astype(jnp.int32)
    cls2 = cls_ctx.reshape(_NUM_CLASSES * _N_CTX, _DIM)
    pre2 = token_prefix.reshape(1, _DIM)
    meta2 = meta_ctx.reshape(_N_META, _DIM)
    suf2 = token_suffix.reshape(_SUFFIX_LEN, _DIM)
    brows = jnp.concatenate([pre2, meta2, suf2], axis=0)
    rep = jnp.broadcast_to(brows[:, None, :], (_NBC, _REP, _DIM))
    rep = jnp.reshape(rep, (_NBC, _REP, _DIM))
    out = _sc_call(lbl, cls2, rep)
    return jnp.transpose(out, (1, 0, 2))


# speedup vs baseline: 1.0518x; 1.0486x over previous
import functools

import jax
import jax.numpy as jnp
from jax import lax
from jax.experimental import pallas as pl
from jax.experimental.pallas import tpu as pltpu
from jax.experimental.pallas import tpu_sc as plsc

_NUM_CLASSES = 56
_N_CTX = 4
_N_META = 16
_SUFFIX_LEN = 60
_DIM = 512
_BATCH = 4096
_ROWS = 1 + _N_CTX + _N_META + _SUFFIX_LEN
_NBC = _ROWS - _N_CTX

_NC = 2
_NS = 16
_NW = _NC * _NS
_BPW = _BATCH // _NW
_LANES = 16

_CCH = 32
_CUN = _N_CTX * (_BPW // _CCH)
_REP = 64
_QTR = 2048
_BUN = _NBC * (_BATCH // _QTR)
_BPWK = 5


def _body(lbl_hbm, cls_hbm, rep_hbm, out_hbm,
          lbl_v, idx_v, cbuf0, cbuf1, bbuf0, bbuf1,
          gs0, gs1, cs0, cs1, ws0, ws1):
    wid = lax.axis_index("s") * _NC + lax.axis_index("c")
    base = wid * _BPW

    pltpu.sync_copy(lbl_hbm.at[pl.ds(base, _BPW)], lbl_v)

    lanes = jax.lax.iota(jnp.int32, _LANES)
    for g in range(_BPW // _LANES):
        lv = lbl_v[pl.ds(g * _LANES, _LANES)]
        for rr in range(_N_CTX):
            plsc.store_scatter(idx_v, [lanes + (rr * _BPW + g * _LANES)],
                               lv * _N_CTX + rr)

    cbufs = (cbuf0, cbuf1)
    gsems = (gs0, gs1)
    csems = (cs0, cs1)

    def cls_gather(i, buf, sem):
        off = pl.multiple_of(i * _CCH, _CCH)
        return pltpu.async_copy(cls_hbm.at[idx_v.at[pl.ds(off, _CCH)]], buf,
                                sem)

    def cls_dst(i):
        row = 1 + i // (_BPW // _CCH)
        boff = base + _CCH * (i % (_BPW // _CCH))
        return out_hbm.at[row, pl.ds(boff, _CCH)]

    for s in range(2):
        cls_gather(s, cbufs[s], gsems[s])
    for i in range(_CUN):
        s = i % 2
        pltpu.make_async_copy(cls_hbm.at[pl.ds(0, _CCH)], cbufs[s],
                              gsems[s]).wait()
        pltpu.async_copy(cbufs[s], cls_dst(i), csems[s])
        if i + 2 < _CUN:
            pltpu.make_async_copy(cbufs[s], cls_dst(i), csems[s]).wait()
            cls_gather(i + 2, cbufs[s], gsems[s])

    bbufs = (bbuf0, bbuf1)
    wsems = (ws0, ws1)
    for t in range(_BPWK):
        s = t % 2
        u = wid + _NW * t
        u = jnp.where(u < _BUN, u, u - _BUN)
        j = u // (_BATCH // _QTR)
        qtr = u % (_BATCH // _QTR)
        row = jnp.where(j == 0, 0, j + _N_CTX)
        if t >= 2:
            for _ in range(_QTR // _REP):
                pltpu.make_async_copy(
                    bbufs[s], out_hbm.at[0, pl.ds(0, _REP)], wsems[s]).wait()
        pltpu.sync_copy(rep_hbm.at[j], bbufs[s])
        for k in range(_QTR // _REP):
            pltpu.async_copy(bbufs[s],
                             out_hbm.at[row, pl.ds(qtr * _QTR + k * _REP,
                                                   _REP)],
                             wsems[s])

    for s in range(2):
        pltpu.make_async_copy(cbufs[s], cls_dst(0), csems[s]).wait()
        for _ in range(_QTR // _REP):
            pltpu.make_async_copy(
                bbufs[s], out_hbm.at[0, pl.ds(0, _REP)], wsems[s]).wait()


_sc_call = functools.partial(
    pl.kernel,
    out_type=jax.ShapeDtypeStruct((_ROWS, _BATCH, _DIM), jnp.float32),
    mesh=plsc.VectorSubcoreMesh(core_axis_name="c", subcore_axis_name="s"),
    compiler_params=pltpu.CompilerParams(needs_layout_passes=False),
    scratch_types=[
        pltpu.VMEM((_BPW,), jnp.int32),
        pltpu.VMEM((_N_CTX * _BPW,), jnp.int32),
        pltpu.VMEM((_CCH, _DIM), jnp.float32),
        pltpu.VMEM((_CCH, _DIM), jnp.float32),
        pltpu.VMEM((_REP, _DIM), jnp.float32),
        pltpu.VMEM((_REP, _DIM), jnp.float32),
        pltpu.SemaphoreType.DMA,
        pltpu.SemaphoreType.DMA,
        pltpu.SemaphoreType.DMA,
        pltpu.SemaphoreType.DMA,
        pltpu.SemaphoreType.DMA,
        pltpu.SemaphoreType.DMA,
    ],
)(_body)


def kernel(label, cls_ctx, meta_ctx, token_prefix, token_suffix):
    lbl = label.astype(jnp.int32)
    cls2 = cls_ctx.reshape(_NUM_CLASSES * _N_CTX, _DIM)
    pre2 = token_prefix.reshape(1, _DIM)
    meta2 = meta_ctx.reshape(_N_META, _DIM)
    suf2 = token_suffix.reshape(_SUFFIX_LEN, _DIM)
    brows = jnp.concatenate([pre2, meta2, suf2], axis=0)
    rep = jnp.broadcast_to(brows[:, None, :], (_NBC, _REP, _DIM))
    rep = jnp.reshape(rep, (_NBC, _REP, _DIM))
    out = _sc_call(lbl, cls2, rep)
    return jnp.transpose(out, (1, 0, 2))
